# baseline (speedup 1.0000x reference)
import jax
import jax.numpy as jnp
from jax.experimental import pallas as pl
from jax.experimental.pallas import tpu as pltpu

_S = 13


def _encoder_body(r_ref, cues_ref, patches_hbm, out_ref, patch_buf, sems):
    r = r_ref[0]
    b = cues_ref.shape[0]
    c = cues_ref.shape[1]
    hw = patches_hbm.shape[1]
    w = 37

    def _copy(nb, slot):
        return pltpu.make_async_copy(
            patches_hbm.at[nb], patch_buf.at[slot], sems.at[slot])

    for i in range(_S - 1):
        _copy(i, i).start()

    def _body(nb, _):
        slot = jax.lax.rem(nb, _S)
        nxt = nb + _S - 1

        @pl.when(nxt < b)
        def _():
            _copy(nxt, jax.lax.rem(nxt, _S)).start()

        _copy(nb, slot).wait()

        cues = cues_ref[nb]
        patches = patch_buf[slot]

        sim = jax.lax.dot_general(
            cues, patches, (((1,), (1,)), ((), ())),
            preferred_element_type=jnp.float32)
        idx = jnp.argmax(sim, axis=1, keepdims=True)
        hh = idx // w
        ww = idx % w

        pos = jax.lax.broadcasted_iota(jnp.int32, (c, hw), 1)
        rowp = pos // w
        colp = pos % w
        inside = (jnp.abs(rowp - hh) <= r) & (jnp.abs(colp - ww) <= r)
        maskf = jnp.where(inside, 1.0, 0.0)
        num = jax.lax.dot_general(
            maskf, patches, (((1,), (0,)), ((), ())),
            preferred_element_type=jnp.float32)

        nrows = jnp.minimum(hh + r, w - 1) - jnp.maximum(hh - r, 0) + 1
        ncols = jnp.minimum(ww + r, w - 1) - jnp.maximum(ww - r, 0) + 1
        cnt = (nrows * ncols).astype(jnp.float32)
        rois = num / cnt

        toks = jnp.concatenate([cues, rois], axis=0)
        nrm = jnp.sqrt(jnp.sum(toks * toks, axis=1, keepdims=True))
        out_ref[nb] = toks / jnp.maximum(nrm, 1e-12)
        return ()

    jax.lax.fori_loop(0, b, _body, ())


def kernel(cls_tok, regs, patches2d, roi_side):
    b, h, w, d = patches2d.shape
    c = 1 + regs.shape[1]
    hw = h * w
    cues = jnp.concatenate([cls_tok[:, None, :], regs], axis=1)
    patches = patches2d.reshape(b, hw, d)
    r = jnp.asarray(roi_side // 2, jnp.int32).reshape(1)

    out = pl.pallas_call(
        _encoder_body,
        in_specs=[
            pl.BlockSpec(memory_space=pltpu.SMEM),
            pl.BlockSpec(memory_space=pltpu.VMEM),
            pl.BlockSpec(memory_space=pl.ANY),
        ],
        out_specs=pl.BlockSpec(memory_space=pltpu.VMEM),
        out_shape=jax.ShapeDtypeStruct((b, 2 * c, d), jnp.float32),
        scratch_shapes=[
            pltpu.VMEM((_S, hw, d), jnp.float32),
            pltpu.SemaphoreType.DMA((_S,)),
        ],
        compiler_params=pltpu.CompilerParams(
            dimension_semantics=(),
            vmem_limit_bytes=100 * 1024 * 1024,
        ),
    )(r, cues, patches)
    return out


# speedup vs baseline: 1.0186x; 1.0186x over previous
import jax
import jax.numpy as jnp
from jax.experimental import pallas as pl
from jax.experimental.pallas import tpu as pltpu

_NB = 4
_CH = 688


def _encoder_body(r_ref, cues_ref, p0_ref, p1_ref, out_ref):
    r = r_ref[0]
    c = cues_ref.shape[1]
    w = 37
    hw = w * w
    tail = hw - _CH

    for nb in range(_NB):
        p1_ref[nb, tail:_CH, :] = jnp.zeros((_CH - tail, 768), jnp.float32)

        cues = cues_ref[nb]
        ch0 = p0_ref[nb]
        ch1 = p1_ref[nb]

        sim0 = jax.lax.dot_general(
            cues, ch0, (((1,), (1,)), ((), ())),
            preferred_element_type=jnp.float32)
        sim1 = jax.lax.dot_general(
            cues, ch1, (((1,), (1,)), ((), ())),
            preferred_element_type=jnp.float32)
        sim = jnp.concatenate([sim0, sim1], axis=1)
        pos2 = jax.lax.broadcasted_iota(jnp.int32, (c, 2 * _CH), 1)
        sim = jnp.where(pos2 < hw, sim, -jnp.inf)
        idx = jnp.argmax(sim, axis=1, keepdims=True)
        hh = idx // w
        ww = idx % w

        num = jnp.zeros((c, 768), jnp.float32)
        for j, ch in enumerate((ch0, ch1)):
            posj = jax.lax.broadcasted_iota(jnp.int32, (c, _CH), 1) + j * _CH
            rowp = posj // w
            colp = posj % w
            inside = (jnp.abs(rowp - hh) <= r) & (jnp.abs(colp - ww) <= r)
            maskf = jnp.where(inside, 1.0, 0.0)
            num = num + jax.lax.dot_general(
                maskf, ch, (((1,), (0,)), ((), ())),
                preferred_element_type=jnp.float32)

        nrows = jnp.minimum(hh + r, w - 1) - jnp.maximum(hh - r, 0) + 1
        ncols = jnp.minimum(ww + r, w - 1) - jnp.maximum(ww - r, 0) + 1
        cnt = (nrows * ncols).astype(jnp.float32)
        rois = num / cnt

        toks = jnp.concatenate([cues, rois], axis=0)
        nrm = jnp.sqrt(jnp.sum(toks * toks, axis=1, keepdims=True))
        out_ref[nb] = toks / jnp.maximum(nrm, 1e-12)


def kernel(cls_tok, regs, patches2d, roi_side):
    b, h, w, d = patches2d.shape
    c = 1 + regs.shape[1]
    hw = h * w
    cues = jnp.concatenate([cls_tok[:, None, :], regs], axis=1)
    patches = patches2d.reshape(b, hw, d)
    r = jnp.asarray(roi_side // 2, jnp.int32).reshape(1)

    out = pl.pallas_call(
        _encoder_body,
        grid=(b // _NB,),
        in_specs=[
            pl.BlockSpec(memory_space=pltpu.SMEM),
            pl.BlockSpec((_NB, c, d), lambda i: (i, 0, 0)),
            pl.BlockSpec((_NB, _CH, d), lambda i: (i, 0, 0)),
            pl.BlockSpec((_NB, _CH, d), lambda i: (i, 1, 0)),
        ],
        out_specs=pl.BlockSpec((_NB, 2 * c, d), lambda i: (i, 0, 0)),
        out_shape=jax.ShapeDtypeStruct((b, 2 * c, d), jnp.float32),
        compiler_params=pltpu.CompilerParams(
            dimension_semantics=("arbitrary",),
            vmem_limit_bytes=100 * 1024 * 1024,
        ),
    )(r, cues, patches, patches)
    return out


# speedup vs baseline: 1.0339x; 1.0150x over previous
import jax
import jax.numpy as jnp
from jax.experimental import pallas as pl
from jax.experimental.pallas import tpu as pltpu

_NB = 4


def _encoder_body(r_ref, cues_ref, patches_ref, out_ref):
    r = r_ref[0]
    c = cues_ref.shape[1]
    hw = patches_ref.shape[1]
    w = 37

    for nb in range(_NB):
        cues = cues_ref[nb]
        patches = patches_ref[nb]

        sim = jax.lax.dot_general(
            cues, patches, (((1,), (1,)), ((), ())),
            preferred_element_type=jnp.float32)
        idx = jnp.argmax(sim, axis=1, keepdims=True)
        hh = idx // w
        ww = idx % w

        pos = jax.lax.broadcasted_iota(jnp.int32, (c, hw), 1)
        rowp = pos // w
        colp = pos % w
        inside = (jnp.abs(rowp - hh) <= r) & (jnp.abs(colp - ww) <= r)
        maskf = jnp.where(inside, 1.0, 0.0)
        num = jax.lax.dot_general(
            maskf, patches, (((1,), (0,)), ((), ())),
            preferred_element_type=jnp.float32)

        nrows = jnp.minimum(hh + r, w - 1) - jnp.maximum(hh - r, 0) + 1
        ncols = jnp.minimum(ww + r, w - 1) - jnp.maximum(ww - r, 0) + 1
        cnt = (nrows * ncols).astype(jnp.float32)
        rois = num / cnt

        toks = jnp.concatenate([cues, rois], axis=0)
        nrm = jnp.sqrt(jnp.sum(toks * toks, axis=1, keepdims=True))
        out_ref[nb] = toks / jnp.maximum(nrm, 1e-12)


def kernel(cls_tok, regs, patches2d, roi_side):
    b, h, w, d = patches2d.shape
    c = 1 + regs.shape[1]
    hw = h * w
    cues = jnp.concatenate([cls_tok[:, None, :], regs], axis=1)
    patches = patches2d.reshape(b, hw, d)
    r = jnp.asarray(roi_side // 2, jnp.int32).reshape(1)

    out = pl.pallas_call(
        _encoder_body,
        grid=(b // _NB,),
        in_specs=[
            pl.BlockSpec(memory_space=pltpu.SMEM),
            pl.BlockSpec((_NB, c, d), lambda i: (i, 0, 0)),
            pl.BlockSpec((_NB, hw, d), lambda i: (i, 0, 0)),
        ],
        out_specs=pl.BlockSpec((_NB, 2 * c, d), lambda i: (i, 0, 0)),
        out_shape=jax.ShapeDtypeStruct((b, 2 * c, d), jnp.float32),
        compiler_params=pltpu.CompilerParams(
            dimension_semantics=("arbitrary",),
            vmem_limit_bytes=100 * 1024 * 1024,
        ),
    )(r, cues, patches)
    return out
